# baseline (speedup 1.0000x reference)
import functools

import jax
import jax.numpy as jnp
from jax import lax
from jax.experimental import pallas as pl
from jax.experimental.pallas import tpu as pltpu
from jax.experimental.pallas import tpu_sc as plsc

_T = 16384
_D = 4096
_E = 64
_K = 8
_TB = 1024
_NW = 32
_NC = 4
_CHT = _T // _NC
_CT = _CHT // _NW
_NG = _CT // 16


def _wquant_body(w_ref, t_ref):
    w = w_ref[...]
    scale = 1.0 / jnp.maximum(jnp.mean(jnp.abs(w)), 1e-5)
    t_ref[...] = jnp.clip(jnp.round(w * scale), -1.0, 1.0) / scale


def _logits_body(x_ref, t_ref, lt_ref):
    x = x_ref[...]
    scale = 127.0 / jnp.maximum(jnp.max(jnp.abs(x), axis=1, keepdims=True), 1e-5)
    y = jnp.round(x * scale) / scale
    lt_ref[...] = jax.lax.dot_general(
        t_ref[...],
        y,
        (((1,), (1,)), ((), ())),
        preferred_element_type=jnp.float32,
    )


def _router_body(lt_ref, fwt_ref, idxt_ref, lt_v, fw_v, idx_v, sem):
    wid = lax.axis_index("s") * 2 + lax.axis_index("c")
    base = wid * _CT
    copies = [
        pltpu.async_copy(
            lt_ref.at[pl.ds(e * _CHT + base, _CT)], lt_v.at[pl.ds(e * _CT, _CT)], sem
        )
        for e in range(_E)
    ]
    for c in copies:
        c.wait()

    ninf16 = jnp.full((16,), -jnp.inf, jnp.float32)
    zi16 = jnp.zeros((16,), jnp.int32)
    one16 = jnp.full((16,), 1, jnp.int32)

    def group_body(g, _):
        g16 = g * 16

        def expert_step(e, carry):
            tv = list(carry[:_K])
            ti = list(carry[_K:])
            cur = lt_v[pl.ds(e * _CT + g16, 16)]
            curi = zi16 + e
            for j in range(_K):
                m = cur > tv[j]
                tv[j], cur = jnp.where(m, cur, tv[j]), jnp.where(m, tv[j], cur)
                ti[j], curi = jnp.where(m, curi, ti[j]), jnp.where(m, ti[j], curi)
            return tuple(tv) + tuple(ti)

        carry = lax.fori_loop(0, _E, expert_step, (ninf16,) * _K + (zi16,) * _K)
        tv = carry[:_K]
        ti = carry[_K:]
        for k in range(_K):
            idx_v[pl.ds(k * _CT + g16, 16)] = ti[k]

        m0 = tv[0]
        thr = tv[_K - 1]
        es = [jnp.exp(v - m0) for v in tv]
        invden = 1.0 / functools.reduce(jnp.add, es)
        gtc = zi16
        for k in range(_K - 1):
            gtc = gtc + jnp.where(tv[k] > thr, one16, zi16)
        allow = _K - gtc

        def fw_step(e, cnt):
            v = lt_v[pl.ds(e * _CT + g16, 16)]
            gt = v > thr
            eq = v == thr
            admit = gt | (eq & (cnt < allow))
            w = jnp.exp(v - m0) * invden
            fw_v[pl.ds(e * _CT + g16, 16)] = jnp.where(admit, w, 0.0)
            return cnt + jnp.where(eq, one16, zi16)

        lax.fori_loop(0, _E, fw_step, zi16)
        return 0

    lax.fori_loop(0, _NG, group_body, 0)

    out = [
        pltpu.async_copy(
            fw_v.at[pl.ds(e * _CT, _CT)], fwt_ref.at[pl.ds(e * _CHT + base, _CT)], sem
        )
        for e in range(_E)
    ]
    out += [
        pltpu.async_copy(
            idx_v.at[pl.ds(k * _CT, _CT)], idxt_ref.at[pl.ds(k * _CHT + base, _CT)], sem
        )
        for k in range(_K)
    ]
    for c in out:
        c.wait()


def kernel(x, W):
    t = pl.pallas_call(
        _wquant_body,
        out_shape=jax.ShapeDtypeStruct((_E, _D), jnp.float32),
    )(W)

    logits_chunk = pl.pallas_call(
        _logits_body,
        grid=(_CHT // _TB,),
        in_specs=[
            pl.BlockSpec((_TB, _D), lambda i: (i, 0)),
            pl.BlockSpec((_E, _D), lambda i: (0, 0)),
        ],
        out_specs=pl.BlockSpec((_E, _TB), lambda i: (0, i)),
        out_shape=jax.ShapeDtypeStruct((_E, _CHT), jnp.float32),
    )

    router = pl.kernel(
        _router_body,
        out_type=(
            jax.ShapeDtypeStruct((_E * _CHT,), jnp.float32),
            jax.ShapeDtypeStruct((_K * _CHT,), jnp.int32),
        ),
        mesh=plsc.VectorSubcoreMesh(core_axis_name="c", subcore_axis_name="s"),
        scratch_types=[
            pltpu.VMEM((_E * _CT,), jnp.float32),
            pltpu.VMEM((_E * _CT,), jnp.float32),
            pltpu.VMEM((_K * _CT,), jnp.int32),
            pltpu.SemaphoreType.DMA,
        ],
    )

    fwts, idxts = [], []
    for c in range(_NC):
        x_c = lax.slice_in_dim(x, c * _CHT, (c + 1) * _CHT, axis=0)
        lt_c = logits_chunk(x_c, t)
        fwt_c, idxt_c = router(lt_c.reshape(_E * _CHT))
        fwts.append(fwt_c.reshape(_E, _CHT))
        idxts.append(idxt_c.reshape(_K, _CHT))
    fw = jnp.concatenate(fwts, axis=1).T
    idx = jnp.concatenate(idxts, axis=1).T
    return fw, idx


# speedup vs baseline: 1.8639x; 1.8639x over previous
import functools

import jax
import jax.numpy as jnp
from jax import lax
from jax.experimental import pallas as pl
from jax.experimental.pallas import tpu as pltpu
from jax.experimental.pallas import tpu_sc as plsc

_T = 16384
_D = 4096
_E = 64
_K = 8
_TB = 1024
_NW = 32
_WT = _T // _NW
_NS = 4
_CT = _WT // _NS
_NG = _CT // 16


def _wquant_body(w_ref, t_ref):
    w = w_ref[...]
    scale = 1.0 / jnp.maximum(jnp.mean(jnp.abs(w)), 1e-5)
    t_ref[...] = jnp.clip(jnp.round(w * scale), -1.0, 1.0) / scale


def _logits_body(x_ref, t_ref, lt_ref):
    x = x_ref[...]
    scale = 127.0 / jnp.maximum(jnp.max(jnp.abs(x), axis=1, keepdims=True), 1e-5)
    y = jnp.round(x * scale) / scale
    lt_ref[...] = jax.lax.dot_general(
        t_ref[...],
        y,
        (((1,), (1,)), ((), ())),
        preferred_element_type=jnp.float32,
    )


def _router_body(lt_ref, fwt_ref, idxt_ref, lt_v, fw_v, idx_v, sem):
    wid = lax.axis_index("s") * 2 + lax.axis_index("c")

    ninf16 = jnp.full((16,), -jnp.inf, jnp.float32)
    zi16 = jnp.zeros((16,), jnp.int32)
    one16 = jnp.full((16,), 1, jnp.int32)

    for s in range(_NS):
        base = wid * _WT + s * _CT
        copies = [
            pltpu.async_copy(
                lt_ref.at[pl.ds(e * _T + base, _CT)],
                lt_v.at[pl.ds(e * _CT, _CT)],
                sem,
            )
            for e in range(_E)
        ]
        for c in copies:
            c.wait()

        def group_body(g, _):
            g16 = g * 16

            def expert_step(e, carry):
                tv = list(carry[:_K])
                ti = list(carry[_K:])
                cur = lt_v[pl.ds(e * _CT + g16, 16)]
                curi = zi16 + e
                for j in range(_K):
                    m = cur > tv[j]
                    tv[j], cur = jnp.where(m, cur, tv[j]), jnp.where(m, tv[j], cur)
                    ti[j], curi = jnp.where(m, curi, ti[j]), jnp.where(m, ti[j], curi)
                return tuple(tv) + tuple(ti)

            carry = lax.fori_loop(0, _E, expert_step, (ninf16,) * _K + (zi16,) * _K)
            tv = carry[:_K]
            ti = carry[_K:]
            for k in range(_K):
                idx_v[pl.ds(k * _CT + g16, 16)] = ti[k]

            m0 = tv[0]
            thr = tv[_K - 1]
            es = [jnp.exp(v - m0) for v in tv]
            invden = 1.0 / functools.reduce(jnp.add, es)
            gtc = zi16
            for k in range(_K - 1):
                gtc = gtc + jnp.where(tv[k] > thr, one16, zi16)
            allow = _K - gtc

            def fw_step(e, cnt):
                v = lt_v[pl.ds(e * _CT + g16, 16)]
                gt = v > thr
                eq = v == thr
                admit = gt | (eq & (cnt < allow))
                w = jnp.exp(v - m0) * invden
                fw_v[pl.ds(e * _CT + g16, 16)] = jnp.where(admit, w, 0.0)
                return cnt + jnp.where(eq, one16, zi16)

            lax.fori_loop(0, _E, fw_step, zi16)
            return 0

        lax.fori_loop(0, _NG, group_body, 0)

        out = [
            pltpu.async_copy(
                fw_v.at[pl.ds(e * _CT, _CT)],
                fwt_ref.at[pl.ds(e * _T + base, _CT)],
                sem,
            )
            for e in range(_E)
        ]
        out += [
            pltpu.async_copy(
                idx_v.at[pl.ds(k * _CT, _CT)],
                idxt_ref.at[pl.ds(k * _T + base, _CT)],
                sem,
            )
            for k in range(_K)
        ]
        for c in out:
            c.wait()


def kernel(x, W):
    t = pl.pallas_call(
        _wquant_body,
        out_shape=jax.ShapeDtypeStruct((_E, _D), jnp.float32),
    )(W)

    lt = pl.pallas_call(
        _logits_body,
        grid=(_T // _TB,),
        in_specs=[
            pl.BlockSpec((_TB, _D), lambda i: (i, 0)),
            pl.BlockSpec((_E, _D), lambda i: (0, 0)),
        ],
        out_specs=pl.BlockSpec((_E, _TB), lambda i: (0, i)),
        out_shape=jax.ShapeDtypeStruct((_E, _T), jnp.float32),
    )(x, t)

    router = pl.kernel(
        _router_body,
        out_type=(
            jax.ShapeDtypeStruct((_E * _T,), jnp.float32),
            jax.ShapeDtypeStruct((_K * _T,), jnp.int32),
        ),
        mesh=plsc.VectorSubcoreMesh(core_axis_name="c", subcore_axis_name="s"),
        scratch_types=[
            pltpu.VMEM((_E * _CT,), jnp.float32),
            pltpu.VMEM((_E * _CT,), jnp.float32),
            pltpu.VMEM((_K * _CT,), jnp.int32),
            pltpu.SemaphoreType.DMA,
        ],
    )

    fwt, idxt = router(lt.reshape(_E * _T))
    fw = fwt.reshape(_E, _T).T
    idx = idxt.reshape(_K, _T).T
    return fw, idx


# speedup vs baseline: 1.8882x; 1.0131x over previous
import functools

import jax
import jax.numpy as jnp
from jax import lax
from jax.experimental import pallas as pl
from jax.experimental.pallas import tpu as pltpu
from jax.experimental.pallas import tpu_sc as plsc

_T = 16384
_D = 4096
_E = 64
_K = 8
_TB = 1024
_NW = 32
_WT = _T // _NW
_NS = 4
_CT = _WT // _NS
_NG = _CT // 16


def _wquant_body(w_ref, t_ref):
    w = w_ref[...]
    scale = 1.0 / jnp.maximum(jnp.mean(jnp.abs(w)), 1e-5)
    t_ref[...] = jnp.clip(jnp.round(w * scale), -1.0, 1.0) / scale


def _logits_body(x_ref, t_ref, lt_ref):
    x = x_ref[...]
    scale = 127.0 / jnp.maximum(jnp.max(jnp.abs(x), axis=1, keepdims=True), 1e-5)
    y = jnp.round(x * scale) / scale
    lt_ref[...] = jax.lax.dot_general(
        t_ref[...],
        y,
        (((1,), (1,)), ((), ())),
        preferred_element_type=jnp.float32,
    )


def _router_body(lt_ref, fwt_ref, idxt_ref,
                 lt_v0, lt_v1, fw_v0, fw_v1, idx_v0, idx_v1, sem_in, sem_out):
    wid = lax.axis_index("s") * 2 + lax.axis_index("c")
    lt_bufs = (lt_v0, lt_v1)
    fw_bufs = (fw_v0, fw_v1)
    idx_bufs = (idx_v0, idx_v1)

    ninf16 = jnp.full((16,), -jnp.inf, jnp.float32)
    zi16 = jnp.zeros((16,), jnp.int32)
    one16 = jnp.full((16,), 1, jnp.int32)

    def issue_in(s):
        base = wid * _WT + s * _CT
        buf = lt_bufs[s % 2]
        return [
            pltpu.async_copy(
                lt_ref.at[pl.ds(e * _T + base, _CT)],
                buf.at[pl.ds(e * _CT, _CT)],
                sem_in,
            )
            for e in range(_E)
        ]

    def issue_out(s):
        base = wid * _WT + s * _CT
        fwb, idb = fw_bufs[s % 2], idx_bufs[s % 2]
        cs = [
            pltpu.async_copy(
                fwb.at[pl.ds(e * _CT, _CT)],
                fwt_ref.at[pl.ds(e * _T + base, _CT)],
                sem_out,
            )
            for e in range(_E)
        ]
        cs += [
            pltpu.async_copy(
                idb.at[pl.ds(k * _CT, _CT)],
                idxt_ref.at[pl.ds(k * _T + base, _CT)],
                sem_out,
            )
            for k in range(_K)
        ]
        return cs

    in_flight = issue_in(0)
    out_flight = None
    for s in range(_NS):
        lt_v = lt_bufs[s % 2]
        fw_v = fw_bufs[s % 2]
        idx_v = idx_bufs[s % 2]
        for c in in_flight:
            c.wait()
        if s + 1 < _NS:
            in_flight = issue_in(s + 1)

        def group_body(g, _):
            g16 = g * 16

            def expert_step(e, carry):
                tv = list(carry[:_K])
                ti = list(carry[_K:])
                cur = lt_v[pl.ds(e * _CT + g16, 16)]
                curi = zi16 + e
                for j in range(_K):
                    m = cur > tv[j]
                    tv[j], cur = jnp.where(m, cur, tv[j]), jnp.where(m, tv[j], cur)
                    ti[j], curi = jnp.where(m, curi, ti[j]), jnp.where(m, ti[j], curi)
                return tuple(tv) + tuple(ti)

            carry = lax.fori_loop(0, _E, expert_step, (ninf16,) * _K + (zi16,) * _K)
            tv = carry[:_K]
            ti = carry[_K:]
            for k in range(_K):
                idx_v[pl.ds(k * _CT + g16, 16)] = ti[k]

            m0 = tv[0]
            thr = tv[_K - 1]
            es = [jnp.exp(v - m0) for v in tv]
            invden = 1.0 / functools.reduce(jnp.add, es)
            gtc = zi16
            for k in range(_K - 1):
                gtc = gtc + jnp.where(tv[k] > thr, one16, zi16)
            allow = _K - gtc

            def fw_step(e, cnt):
                v = lt_v[pl.ds(e * _CT + g16, 16)]
                gt = v > thr
                eq = v == thr
                admit = gt | (eq & (cnt < allow))
                w = jnp.exp(v - m0) * invden
                fw_v[pl.ds(e * _CT + g16, 16)] = jnp.where(admit, w, 0.0)
                return cnt + jnp.where(eq, one16, zi16)

            lax.fori_loop(0, _E, fw_step, zi16)
            return 0

        lax.fori_loop(0, _NG, group_body, 0)

        if out_flight is not None:
            for c in out_flight:
                c.wait()
        out_flight = issue_out(s)

    for c in out_flight:
        c.wait()


def kernel(x, W):
    t = pl.pallas_call(
        _wquant_body,
        out_shape=jax.ShapeDtypeStruct((_E, _D), jnp.float32),
    )(W)

    lt = pl.pallas_call(
        _logits_body,
        grid=(_T // _TB,),
        in_specs=[
            pl.BlockSpec((_TB, _D), lambda i: (i, 0)),
            pl.BlockSpec((_E, _D), lambda i: (0, 0)),
        ],
        out_specs=pl.BlockSpec((_E, _TB), lambda i: (0, i)),
        out_shape=jax.ShapeDtypeStruct((_E, _T), jnp.float32),
    )(x, t)

    router = pl.kernel(
        _router_body,
        out_type=(
            jax.ShapeDtypeStruct((_E * _T,), jnp.float32),
            jax.ShapeDtypeStruct((_K * _T,), jnp.int32),
        ),
        mesh=plsc.VectorSubcoreMesh(core_axis_name="c", subcore_axis_name="s"),
        scratch_types=[
            pltpu.VMEM((_E * _CT,), jnp.float32),
            pltpu.VMEM((_E * _CT,), jnp.float32),
            pltpu.VMEM((_E * _CT,), jnp.float32),
            pltpu.VMEM((_E * _CT,), jnp.float32),
            pltpu.VMEM((_K * _CT,), jnp.int32),
            pltpu.VMEM((_K * _CT,), jnp.int32),
            pltpu.SemaphoreType.DMA,
            pltpu.SemaphoreType.DMA,
        ],
    )

    fwt, idxt = router(lt.reshape(_E * _T))
    fw = fwt.reshape(_E, _T).T
    idx = idxt.reshape(_K, _T).T
    return fw, idx
